# Initial kernel scaffold; baseline (speedup 1.0000x reference)
import jax
import jax.numpy as jnp
from jax.experimental import pallas as pl


def kernel(adj_matrices, node_features, W1, b1, W2, b2):
    raise NotImplementedError("write your pallas kernel here")


# speedup vs baseline: 1.9038x; 1.9038x over previous
import jax
import jax.numpy as jnp
from jax.experimental import pallas as pl
from jax.experimental.pallas import tpu as pltpu


def _gnn_kernel(adj_ref, xT_ref, W1T_ref, b1_ref, W2T_ref, b2_ref, out_ref):
    n = adj_ref.shape[1]
    adj = adj_ref[0]
    r = jax.lax.broadcasted_iota(jnp.int32, (n, n), 0)
    c = jax.lax.broadcasted_iota(jnp.int32, (n, n), 1)
    ah = jnp.where(r == c, 1.0, adj)
    deg = jnp.sum(ah, axis=0, keepdims=True)
    dinv = jax.lax.rsqrt(deg)
    ah_bf = ah.astype(jnp.bfloat16)

    xT = xT_ref[0]
    q1 = jnp.dot(W1T_ref[...], xT, preferred_element_type=jnp.float32)
    m1 = (q1 * dinv).astype(jnp.bfloat16)
    y1 = jnp.dot(m1, ah_bf, preferred_element_type=jnp.float32)
    h1 = jnp.maximum(y1 * dinv + b1_ref[...], 0.0)

    q2 = jnp.dot(W2T_ref[...], h1, preferred_element_type=jnp.float32)
    m2 = (q2 * dinv).astype(jnp.bfloat16)
    y2 = jnp.dot(m2, ah_bf, preferred_element_type=jnp.float32)
    h2 = jnp.maximum(y2 * dinv + b2_ref[...], 0.0)

    out_ref[0, 0, :] = jnp.mean(h2, axis=1)


def kernel(adj_matrices, node_features, W1, b1, W2, b2):
    B, N, Dd = node_features.shape
    H = W1.shape[1]
    xT = jnp.transpose(node_features, (0, 2, 1))
    W1T = W1.T
    W2T = W2.T
    b1c = b1[:, None]
    b2c = b2[:, None]

    return pl.pallas_call(
        _gnn_kernel,
        grid=(B,),
        in_specs=[
            pl.BlockSpec((1, N, N), lambda b: (b, 0, 0)),
            pl.BlockSpec((1, Dd, N), lambda b: (b, 0, 0)),
            pl.BlockSpec((H, Dd), lambda b: (0, 0)),
            pl.BlockSpec((H, 1), lambda b: (0, 0)),
            pl.BlockSpec((H, H), lambda b: (0, 0)),
            pl.BlockSpec((H, 1), lambda b: (0, 0)),
        ],
        out_specs=pl.BlockSpec((1, 1, H), lambda b: (b, 0, 0)),
        out_shape=jax.ShapeDtypeStruct((B, 1, H), jnp.float32),
        compiler_params=pltpu.CompilerParams(
            dimension_semantics=("parallel",),
        ),
    )(adj_matrices, xT, W1T, b1c, W2T, b2c)[:, 0, :]
